# baseline (speedup 1.0000x reference)
import jax
import jax.numpy as jnp
from jax.experimental import pallas as pl
from jax.experimental.pallas import tpu as pltpu

_NBUF = 8


def _body(x_hbm, w_ref, b_ref, o_hbm, xbuf, ybuf, insem, outsem):
    B, T, K, S = x_hbm.shape

    def start_in(i, slot):
        pltpu.make_async_copy(
            x_hbm.at[i], xbuf.at[slot], insem.at[slot]
        ).start()

    def wait_in(slot):
        pltpu.make_async_copy(
            x_hbm.at[0], xbuf.at[slot], insem.at[slot]
        ).wait()

    def start_out(i, slot):
        pltpu.make_async_copy(
            ybuf.at[slot], o_hbm.at[i], outsem.at[slot]
        ).start()

    def wait_out(slot):
        pltpu.make_async_copy(
            ybuf.at[slot], o_hbm.at[0], outsem.at[slot]
        ).wait()

    for s in range(_NBUF):
        start_in(s, s)

    w = w_ref[...]
    bb = b_ref[...]

    def step(i, _):
        slot = jax.lax.rem(i, _NBUF)
        wait_in(slot)

        @pl.when(i >= _NBUF)
        def _():
            wait_out(slot)

        m = jnp.mean(xbuf[slot], axis=1)
        y = jnp.dot(m, w, preferred_element_type=jnp.float32) + bb
        ybuf[slot] = jnp.broadcast_to(y[:, None, :], ybuf.shape[1:])
        start_out(i, slot)

        @pl.when(i + _NBUF < B)
        def _():
            start_in(i + _NBUF, slot)

        return 0

    jax.lax.fori_loop(0, B, step, 0)

    for s in range(_NBUF):
        wait_out(s)


def kernel(x, W, b):
    B, T, K, S = x.shape
    Z = W.shape[1]
    out = pl.pallas_call(
        _body,
        in_specs=[
            pl.BlockSpec(memory_space=pl.ANY),
            pl.BlockSpec(memory_space=pltpu.VMEM),
            pl.BlockSpec(memory_space=pltpu.VMEM),
        ],
        out_specs=pl.BlockSpec(memory_space=pl.ANY),
        out_shape=jax.ShapeDtypeStruct((B, T, K, Z), jnp.float32),
        scratch_shapes=[
            pltpu.VMEM((_NBUF, T, K, S), jnp.float32),
            pltpu.VMEM((_NBUF, T, K, Z), jnp.float32),
            pltpu.SemaphoreType.DMA((_NBUF,)),
            pltpu.SemaphoreType.DMA((_NBUF,)),
        ],
    )(x, W, b.reshape(1, Z))
    return out


# speedup vs baseline: 1.3006x; 1.3006x over previous
import jax
import jax.numpy as jnp
from jax.experimental import pallas as pl
from jax.experimental.pallas import tpu as pltpu

_NBUF = 8


def _body(x_hbm, w_ref, b_ref, o_hbm, xbuf, ybuf, insem, outsem):
    B, T, K, S = x_hbm.shape

    def start_in(i, slot):
        pltpu.make_async_copy(
            x_hbm.at[i], xbuf.at[slot], insem.at[slot]
        ).start()

    def wait_in(slot):
        pltpu.make_async_copy(
            x_hbm.at[0], xbuf.at[slot], insem.at[slot]
        ).wait()

    def start_out(i, slot):
        pltpu.make_async_copy(
            ybuf.at[slot], o_hbm.at[i], outsem.at[slot]
        ).start()

    def wait_out(slot):
        pltpu.make_async_copy(
            ybuf.at[slot], o_hbm.at[0], outsem.at[slot]
        ).wait()

    for s in range(_NBUF):
        start_in(s, s)

    w = w_ref[...]
    bb = b_ref[...]

    def step(i, _):
        slot = jax.lax.rem(i, _NBUF)
        wait_in(slot)

        @pl.when(i >= _NBUF)
        def _():
            wait_out(slot)

        m = jnp.mean(xbuf[slot], axis=1)
        ybuf[slot] = (
            jnp.dot(m, w, preferred_element_type=jnp.float32) + bb
        )
        start_out(i, slot)

        @pl.when(i + _NBUF < B)
        def _():
            start_in(i + _NBUF, slot)

        return 0

    jax.lax.fori_loop(0, B, step, 0)

    for s in range(_NBUF):
        wait_out(s)


def kernel(x, W, b):
    B, T, K, S = x.shape
    Z = W.shape[1]
    Wt = jnp.tile(W, (1, K))
    bt = jnp.tile(b, K).reshape(1, K * Z)
    out = pl.pallas_call(
        _body,
        in_specs=[
            pl.BlockSpec(memory_space=pl.ANY),
            pl.BlockSpec(memory_space=pltpu.VMEM),
            pl.BlockSpec(memory_space=pltpu.VMEM),
        ],
        out_specs=pl.BlockSpec(memory_space=pl.ANY),
        out_shape=jax.ShapeDtypeStruct((B, T, K * Z), jnp.float32),
        scratch_shapes=[
            pltpu.VMEM((_NBUF, T, K, S), jnp.float32),
            pltpu.VMEM((_NBUF, T, K * Z), jnp.float32),
            pltpu.SemaphoreType.DMA((_NBUF,)),
            pltpu.SemaphoreType.DMA((_NBUF,)),
        ],
    )(x, Wt, bt)
    return out.reshape(B, T, K, Z)
